# Initial kernel scaffold; baseline (speedup 1.0000x reference)
import jax
import jax.numpy as jnp
from jax.experimental import pallas as pl


def kernel(x, y):
    raise NotImplementedError("write your pallas kernel here")


# speedup vs baseline: 2.0921x; 2.0921x over previous
import jax
import jax.numpy as jnp
from jax.experimental import pallas as pl
from jax.experimental.pallas import tpu as pltpu

_BINS = 10
_DELTA = 0.1
_SIGMA = 150.0
_LOG2E = 1.4426950408889634
_K2 = _SIGMA * _LOG2E
_SHIFT = 108.0
_EDGE_C = [float(2.0 ** (_K2 * _DELTA * i - _SHIFT)) for i in range(_BINS + 1)]

_ROWS = 48
_COLS = 512 * 512
_RBLK = 24
_PBLK = 8192
_KSTEPS = _COLS // _PBLK
_ACC_LANES = 16


def _edge_sums_kernel(x_ref, y_ref, acc_ref):
    k = pl.program_id(1)

    @pl.when(k == 0)
    def _init():
        acc_ref[...] = jnp.zeros_like(acc_ref)

    xb = x_ref[...]
    yb = y_ref[...]
    px = jnp.exp2(_SHIFT - _K2 * xb)
    py = jnp.exp2(_SHIFT - _K2 * yb)
    cols = []
    for i in range(_BINS + 1):
        c = _EDGE_C[i]
        sig_x = 1.0 / (1.0 + c * px)
        sig_y = 1.0 / (1.0 + c * py)
        cols.append(jnp.sum(sig_x - sig_y, axis=1, keepdims=True))
    for _ in range(_ACC_LANES - (_BINS + 1)):
        cols.append(jnp.zeros_like(cols[0]))
    acc_ref[...] += jnp.concatenate(cols, axis=1)


def _loss_kernel(acc_ref, out_ref):
    u = acc_ref[...]
    d = u[:, 0:_BINS] - u[:, 1:_BINS + 1]
    total = jnp.sum(jnp.abs(d), axis=(0, 1), keepdims=True)
    out_ref[...] = total * (0.0001 / (_BINS * 16))


def kernel(x, y):
    xr = x.reshape(_ROWS, _COLS)
    yr = y.reshape(_ROWS, _COLS)

    acc = pl.pallas_call(
        _edge_sums_kernel,
        grid=(_ROWS // _RBLK, _KSTEPS),
        in_specs=[
            pl.BlockSpec((_RBLK, _PBLK), lambda g, k: (g, k)),
            pl.BlockSpec((_RBLK, _PBLK), lambda g, k: (g, k)),
        ],
        out_specs=pl.BlockSpec((_RBLK, _ACC_LANES), lambda g, k: (g, 0)),
        out_shape=jax.ShapeDtypeStruct((_ROWS, _ACC_LANES), jnp.float32),
        compiler_params=pltpu.CompilerParams(
            dimension_semantics=("parallel", "arbitrary"),
        ),
    )(xr, yr)

    out = pl.pallas_call(
        _loss_kernel,
        out_shape=jax.ShapeDtypeStruct((1, 1), jnp.float32),
    )(acc)
    return out[0, 0]


# speedup vs baseline: 3.0968x; 1.4803x over previous
import jax
import jax.numpy as jnp
from jax.experimental import pallas as pl
from jax.experimental.pallas import tpu as pltpu

_BINS = 10
_DELTA = 0.1
_SIGMA = 150.0
_LOG2E = 1.4426950408889634
_K2 = _SIGMA * _LOG2E
_SHIFT = 108.0
_EDGE_C = [float(2.0 ** (_K2 * _DELTA * i - _SHIFT)) for i in range(_BINS + 1)]

_ROWS = 48
_H = 512
_W = 512
_RBLK = 24
_HBLK = 16
_KSTEPS = _H // _HBLK
_ACC_LANES = 16


def _edge_sums_kernel(x_ref, y_ref, acc_ref):
    k = pl.program_id(1)

    @pl.when(k == 0)
    def _init():
        acc_ref[...] = jnp.zeros_like(acc_ref)

    xb = x_ref[...]
    yb = y_ref[...]
    px = jnp.exp2(_SHIFT - _K2 * xb)
    py = jnp.exp2(_SHIFT - _K2 * yb)
    cols = []
    for i in range(_BINS + 1):
        c = _EDGE_C[i]
        sig_x = 1.0 / (1.0 + c * px)
        sig_y = 1.0 / (1.0 + c * py)
        cols.append(jnp.sum(sig_x - sig_y, axis=(1, 2), keepdims=True)[:, 0, :])
    for _ in range(_ACC_LANES - (_BINS + 1)):
        cols.append(jnp.zeros_like(cols[0]))
    acc_ref[...] += jnp.concatenate(cols, axis=1)


def _loss_kernel(acc_ref, out_ref):
    u = acc_ref[...]
    d = u[:, 0:_BINS] - u[:, 1:_BINS + 1]
    total = jnp.sum(jnp.abs(d), axis=(0, 1), keepdims=True)
    out_ref[...] = total * (0.0001 / (_BINS * 16))


def kernel(x, y):
    xr = x.reshape(_ROWS, _H, _W)
    yr = y.reshape(_ROWS, _H, _W)

    acc = pl.pallas_call(
        _edge_sums_kernel,
        grid=(_ROWS // _RBLK, _KSTEPS),
        in_specs=[
            pl.BlockSpec((_RBLK, _HBLK, _W), lambda g, k: (g, k, 0)),
            pl.BlockSpec((_RBLK, _HBLK, _W), lambda g, k: (g, k, 0)),
        ],
        out_specs=pl.BlockSpec((_RBLK, _ACC_LANES), lambda g, k: (g, 0)),
        out_shape=jax.ShapeDtypeStruct((_ROWS, _ACC_LANES), jnp.float32),
        compiler_params=pltpu.CompilerParams(
            dimension_semantics=("parallel", "arbitrary"),
        ),
    )(xr, yr)

    out = pl.pallas_call(
        _loss_kernel,
        out_shape=jax.ShapeDtypeStruct((1, 1), jnp.float32),
    )(acc)
    return out[0, 0]


# speedup vs baseline: 3.2214x; 1.0402x over previous
import functools

import jax
import jax.numpy as jnp
from jax.experimental import pallas as pl
from jax.experimental.pallas import tpu as pltpu

_BINS = 10
_DELTA = 0.1
_SIGMA = 150.0
_LOG2E = 1.4426950408889634
_K2 = _SIGMA * _LOG2E
_SHIFT = 108.0
_EDGE_C = [float(2.0 ** (_K2 * _DELTA * i - _SHIFT)) for i in range(_BINS + 1)]

_ROWS = 48
_H = 512
_W = 512
_NCORES = 2
_RBLK = _ROWS // _NCORES
_HBLK = 16
_KSTEPS = _H // _HBLK
_ACC_LANES = 16


def _edge_sums_body(acc_vmem, x_blk, y_blk):
    xb = x_blk[...]
    yb = y_blk[...]
    px = jnp.exp2(_SHIFT - _K2 * xb).astype(jnp.bfloat16)
    py = jnp.exp2(_SHIFT - _K2 * yb).astype(jnp.bfloat16)
    one = jnp.bfloat16(1.0)
    cols = []
    for i in range(_BINS + 1):
        c = jnp.bfloat16(_EDGE_C[i])
        sig_x = one / (one + c * px)
        sig_y = one / (one + c * py)
        d = sig_x - sig_y
        q = (d[:, :, 0:128] + d[:, :, 128:256]) + (d[:, :, 256:384] + d[:, :, 384:512])
        q32 = q.astype(jnp.float32)
        cols.append(jnp.sum(q32, axis=(1, 2), keepdims=True)[:, 0, :])
    for _ in range(_ACC_LANES - (_BINS + 1)):
        cols.append(jnp.zeros_like(cols[0]))
    acc_vmem[...] += jnp.concatenate(cols, axis=1)


def _core_worker(x_ref, y_ref, acc_ref, acc_vmem, sem):
    core = jax.lax.axis_index("core")
    acc_vmem[...] = jnp.zeros((_RBLK, _ACC_LANES), jnp.float32)
    pltpu.emit_pipeline(
        functools.partial(_edge_sums_body, acc_vmem),
        grid=(_NCORES, _KSTEPS),
        in_specs=[
            pl.BlockSpec((_RBLK, _HBLK, _W), lambda g, k: (g, k, 0)),
            pl.BlockSpec((_RBLK, _HBLK, _W), lambda g, k: (g, k, 0)),
        ],
        core_axis_name="core",
        dimension_semantics=(pltpu.PARALLEL, pltpu.ARBITRARY),
    )(x_ref, y_ref)
    cp = pltpu.make_async_copy(
        acc_vmem, acc_ref.at[pl.ds(core * _RBLK, _RBLK)], sem)
    cp.start()
    cp.wait()


def _loss_kernel(acc_ref, out_ref):
    u = acc_ref[...]
    d = u[:, 0:_BINS] - u[:, 1:_BINS + 1]
    total = jnp.sum(jnp.abs(d), axis=(0, 1), keepdims=True)
    out_ref[...] = total * (0.0001 / (_BINS * 16))


def kernel(x, y):
    xr = x.reshape(_ROWS, _H, _W)
    yr = y.reshape(_ROWS, _H, _W)

    mesh = pltpu.create_tensorcore_mesh("core", num_cores=_NCORES)

    def run(refs):
        x_ref, y_ref, acc_ref = refs

        @pl.core_map(mesh)
        def _():
            pl.run_scoped(
                functools.partial(_core_worker, x_ref, y_ref, acc_ref),
                pltpu.VMEM((_RBLK, _ACC_LANES), jnp.float32),
                pltpu.SemaphoreType.DMA,
            )

    _, _, acc = pl.run_state(run)(
        (xr, yr, jnp.zeros((_ROWS, _ACC_LANES), jnp.float32)))

    out = pl.pallas_call(
        _loss_kernel,
        out_shape=jax.ShapeDtypeStruct((1, 1), jnp.float32),
    )(acc)
    return out[0, 0]
